# baseline (speedup 1.0000x reference)
import numpy as np
import jax
import jax.numpy as jnp
from jax.experimental import pallas as pl
from jax.experimental.pallas import tpu as pltpu

_FEATURES_OUT = 128
_BN_EPS = 1e-5


def _head_body(x_ref,
               p_ref,
               w_ref,
               b_ref,
               s2_ref,
               sh2_ref,
               o_ref,
               *, groups, group_lanes):
    parts = []
    for g in range(groups):
        xg = x_ref[:, g * group_lanes:(g + 1) * group_lanes]
        parts.append(jnp.dot(xg, p_ref[...],
                             preferred_element_type=jnp.float32))
    s = jnp.concatenate(parts, axis=-1)
    y = jnp.dot(s, w_ref[...], preferred_element_type=jnp.float32)
    y = jnp.maximum(y + b_ref[...], 0.0)
    o_ref[...] = (y * s2_ref[...] + sh2_ref[...]).astype(o_ref.dtype)


@jax.jit
def _head(feat_nchw, params):
    n, c, h, w = feat_nchw.shape
    hw = h * w
    assert c % 128 == 0
    groups = c // 128
    group_lanes = 128 * hw

    feat = feat_nchw.reshape(n, c * hw)

    s1 = params["bn1_gamma"] * jax.lax.rsqrt(params["bn1_var"] + _BN_EPS)
    w_fold = (s1.reshape(c, 1) * params["lin_w_t"]) * (1.0 / hw)
    b_fold = ((params["bn1_beta"] - params["bn1_mean"] * s1)
              @ params["lin_w_t"] + params["lin_b"])
    s2 = params["bn2_gamma"] * jax.lax.rsqrt(params["bn2_var"] + _BN_EPS)
    sh2 = params["bn2_beta"] - params["bn2_mean"] * s2

    pool = jnp.asarray(np.repeat(np.eye(128, dtype=np.float32), hw, axis=0))

    tn = min(64, n)
    grid = (pl.cdiv(n, tn),)

    out = pl.pallas_call(
        lambda *refs: _head_body(*refs, groups=groups,
                                 group_lanes=group_lanes),
        out_shape=jax.ShapeDtypeStruct((n, _FEATURES_OUT), jnp.float32),
        grid=grid,
        in_specs=[
            pl.BlockSpec((tn, c * hw), lambda i: (i, 0)),
            pl.BlockSpec((group_lanes, 128), lambda i: (0, 0)),
            pl.BlockSpec((c, _FEATURES_OUT), lambda i: (0, 0)),
            pl.BlockSpec((1, _FEATURES_OUT), lambda i: (0, 0)),
            pl.BlockSpec((1, _FEATURES_OUT), lambda i: (0, 0)),
            pl.BlockSpec((1, _FEATURES_OUT), lambda i: (0, 0)),
        ],
        out_specs=pl.BlockSpec((tn, _FEATURES_OUT), lambda i: (i, 0)),
        compiler_params=pltpu.CompilerParams(
            dimension_semantics=("parallel",),
            vmem_limit_bytes=48 * 1024 * 1024,
        ),
    )(feat, pool, w_fold, b_fold, s2, sh2)
    return out


def kernel(feat_nchw, bn1_gamma, bn1_beta, bn1_mean, bn1_var,
           lin_w_t, lin_b, bn2_gamma, bn2_beta, bn2_mean, bn2_var):
    params = {
        "bn1_gamma": bn1_gamma,
        "bn1_beta": bn1_beta,
        "bn1_mean": bn1_mean,
        "bn1_var": bn1_var,
        "lin_w_t": lin_w_t,
        "lin_b": lin_b,
        "bn2_gamma": bn2_gamma,
        "bn2_beta": bn2_beta,
        "bn2_mean": bn2_mean,
        "bn2_var": bn2_var,
    }
    return _head(feat_nchw, params)


# speedup vs baseline: 8.9161x; 8.9161x over previous
import jax
import jax.numpy as jnp
from jax.experimental import pallas as pl
from jax.experimental.pallas import tpu as pltpu

_FEATURES_OUT = 128
_BN_EPS = 1e-5


def _head_body(x_ref,
               w_ref,
               b_ref,
               s2_ref,
               sh2_ref,
               o_ref):
    s = jnp.sum(x_ref[...], axis=0)
    y = jnp.dot(s, w_ref[...], preferred_element_type=jnp.float32)
    y = jnp.maximum(y + b_ref[...], 0.0)
    o_ref[...] = (y * s2_ref[...] + sh2_ref[...]).astype(o_ref.dtype)


@jax.jit
def _head(feat_nchw, params):
    n, c, h, w = feat_nchw.shape
    hw = h * w

    feat = feat_nchw.transpose(2, 3, 0, 1).reshape(hw, n, c)

    s1 = params["bn1_gamma"] * jax.lax.rsqrt(params["bn1_var"] + _BN_EPS)
    w_fold = (s1.reshape(c, 1) * params["lin_w_t"]) * (1.0 / hw)
    b_fold = ((params["bn1_beta"] - params["bn1_mean"] * s1)
              @ params["lin_w_t"] + params["lin_b"])
    s2 = params["bn2_gamma"] * jax.lax.rsqrt(params["bn2_var"] + _BN_EPS)
    sh2 = params["bn2_beta"] - params["bn2_mean"] * s2

    tn = min(64, n)
    grid = (pl.cdiv(n, tn),)

    out = pl.pallas_call(
        _head_body,
        out_shape=jax.ShapeDtypeStruct((n, _FEATURES_OUT), jnp.float32),
        grid=grid,
        in_specs=[
            pl.BlockSpec((hw, tn, c), lambda i: (0, i, 0)),
            pl.BlockSpec((c, _FEATURES_OUT), lambda i: (0, 0)),
            pl.BlockSpec((1, _FEATURES_OUT), lambda i: (0, 0)),
            pl.BlockSpec((1, _FEATURES_OUT), lambda i: (0, 0)),
            pl.BlockSpec((1, _FEATURES_OUT), lambda i: (0, 0)),
        ],
        out_specs=pl.BlockSpec((tn, _FEATURES_OUT), lambda i: (i, 0)),
        compiler_params=pltpu.CompilerParams(
            dimension_semantics=("parallel",),
            vmem_limit_bytes=48 * 1024 * 1024,
        ),
    )(feat, w_fold, b_fold, s2, sh2)
    return out


def kernel(feat_nchw, bn1_gamma, bn1_beta, bn1_mean, bn1_var,
           lin_w_t, lin_b, bn2_gamma, bn2_beta, bn2_mean, bn2_var):
    params = {
        "bn1_gamma": bn1_gamma,
        "bn1_beta": bn1_beta,
        "bn1_mean": bn1_mean,
        "bn1_var": bn1_var,
        "lin_w_t": lin_w_t,
        "lin_b": lin_b,
        "bn2_gamma": bn2_gamma,
        "bn2_beta": bn2_beta,
        "bn2_mean": bn2_mean,
        "bn2_var": bn2_var,
    }
    return _head(feat_nchw, params)


# speedup vs baseline: 9.0249x; 1.0122x over previous
import jax
import jax.numpy as jnp
from jax.experimental import pallas as pl
from jax.experimental.pallas import tpu as pltpu

_FEATURES_OUT = 128
_BN_EPS = 1e-5


def _head_body(x_ref,
               w_ref,
               b_ref,
               s2_ref,
               sh2_ref,
               o_ref):
    s = jnp.sum(x_ref[...], axis=0)
    y = jnp.dot(s, w_ref[...], preferred_element_type=jnp.float32)
    y = jnp.maximum(y + b_ref[...], 0.0)
    o_ref[...] = (y * s2_ref[...] + sh2_ref[...]).astype(o_ref.dtype)


@jax.jit
def _head(feat_nchw, params):
    n, c, h, w = feat_nchw.shape
    hw = h * w

    feat = feat_nchw.transpose(2, 3, 0, 1).reshape(hw, n, c)

    s1 = params["bn1_gamma"] * jax.lax.rsqrt(params["bn1_var"] + _BN_EPS)
    w_fold = (s1.reshape(c, 1) * params["lin_w_t"]) * (1.0 / hw)
    b_fold = ((params["bn1_beta"] - params["bn1_mean"] * s1)
              @ params["lin_w_t"] + params["lin_b"])
    s2 = params["bn2_gamma"] * jax.lax.rsqrt(params["bn2_var"] + _BN_EPS)
    sh2 = params["bn2_beta"] - params["bn2_mean"] * s2

    tn = min(32, n)
    grid = (pl.cdiv(n, tn),)

    out = pl.pallas_call(
        _head_body,
        out_shape=jax.ShapeDtypeStruct((n, _FEATURES_OUT), jnp.float32),
        grid=grid,
        in_specs=[
            pl.BlockSpec((hw, tn, c), lambda i: (0, i, 0)),
            pl.BlockSpec((c, _FEATURES_OUT), lambda i: (0, 0)),
            pl.BlockSpec((1, _FEATURES_OUT), lambda i: (0, 0)),
            pl.BlockSpec((1, _FEATURES_OUT), lambda i: (0, 0)),
            pl.BlockSpec((1, _FEATURES_OUT), lambda i: (0, 0)),
        ],
        out_specs=pl.BlockSpec((tn, _FEATURES_OUT), lambda i: (i, 0)),
        compiler_params=pltpu.CompilerParams(
            dimension_semantics=("parallel",),
            vmem_limit_bytes=48 * 1024 * 1024,
        ),
    )(feat, w_fold, b_fold, s2, sh2)
    return out


def kernel(feat_nchw, bn1_gamma, bn1_beta, bn1_mean, bn1_var,
           lin_w_t, lin_b, bn2_gamma, bn2_beta, bn2_mean, bn2_var):
    params = {
        "bn1_gamma": bn1_gamma,
        "bn1_beta": bn1_beta,
        "bn1_mean": bn1_mean,
        "bn1_var": bn1_var,
        "lin_w_t": lin_w_t,
        "lin_b": lin_b,
        "bn2_gamma": bn2_gamma,
        "bn2_beta": bn2_beta,
        "bn2_mean": bn2_mean,
        "bn2_var": bn2_var,
    }
    return _head(feat_nchw, params)
